# Initial kernel scaffold; baseline (speedup 1.0000x reference)
import jax
import jax.numpy as jnp
from jax.experimental import pallas as pl


def kernel(predicted_target, target):
    raise NotImplementedError("write your pallas kernel here")


# speedup vs baseline: 1.7998x; 1.7998x over previous
import functools
import math

import jax
import jax.numpy as jnp
from jax import lax
from jax.experimental import pallas as pl
from jax.experimental.pallas import tpu as pltpu
from jax.experimental.pallas import tpu_sc as plsc

_V = 100000
_N = 1024
_PAD = 0
_SMOOTHING = 0.1
_WB = 2048
_GRID = (_V + _WB - 1) // _WB

_NW = 32
_RPW = _N // _NW
_LANES = 16


def _tc_body(tgt_ref, x_ref, t_ref, z_ref, n1_ref):
    k = pl.program_id(0)
    x = x_ref[...]
    m = (tgt_ref[...] != _PAD).astype(jnp.float32)
    col = jax.lax.broadcasted_iota(jnp.int32, (_N, _WB), 1) + k * _WB
    xv = jnp.where(col < _V, x, 0.0)
    rowp = jnp.sum(xv, axis=1, keepdims=True)
    part = jnp.sum(rowp * m)

    @pl.when(k == 0)
    def _():
        t_ref[0, 0] = 0.0
        z_ref[0, 0] = jnp.sum(x[:, 0:1] * m)
        n1_ref[0, 0] = jnp.sum(m)

    t_ref[0, 0] += part


def _tc_sums(predicted_target, target_col):
    scalar = jax.ShapeDtypeStruct((1, 1), jnp.float32)
    smem = pl.BlockSpec(memory_space=pltpu.MemorySpace.SMEM)
    return pl.pallas_call(
        _tc_body,
        grid=(_GRID,),
        in_specs=[
            pl.BlockSpec((_N, 1), lambda k: (0, 0)),
            pl.BlockSpec((_N, _WB), lambda k: (0, k)),
        ],
        out_specs=[smem, smem, smem],
        out_shape=[scalar, scalar, scalar],
        compiler_params=pltpu.CompilerParams(
            dimension_semantics=("arbitrary",)),
    )(target_col, predicted_target)


_WIN = 128


def _sc_gather_kernel(pt_hbm, tgt_hbm, out_hbm, t_v, c_v, blk_v, acc_v):
    wid = lax.axis_index("s") * 2 + lax.axis_index("c")
    base = wid * _RPW
    pltpu.sync_copy(tgt_hbm.at[pl.ds(base, _RPW)], t_v.at[pl.ds(0, _RPW)])

    for h in range(_RPW // _LANES):
        t = t_v[pl.ds(h * _LANES, _LANES)]
        c_v[pl.ds(h * _LANES, _LANES)] = t & (-128)
        nz = lax.shift_right_logical(t | (0 - t), 31)
        t_v[pl.ds(h * _LANES, _LANES)] = (t & 127) + (nz << 7) - 128

    lane = lax.iota(jnp.int32, _LANES)
    acc_v[...] = jnp.zeros((_LANES,), jnp.float32)

    def body(j, carry):
        o = t_v[pl.ds(j, _LANES)][0]
        c = pl.multiple_of(c_v[pl.ds(j, _LANES)][0], 128)
        r0 = pl.multiple_of(base + (j & (-8)), 8)
        pltpu.sync_copy(pt_hbm.at[pl.ds(r0, 8), pl.ds(c, _WIN)], blk_v)
        rm = j & 7
        off = jnp.full((_LANES,), o, jnp.int32)
        zero = jnp.zeros((_LANES,), jnp.float32)
        val = zero
        for h in range(_WIN // _LANES):
            sel = (lane + h * _LANES) == off
            val = val + jnp.where(sel, blk_v[rm, pl.ds(h * _LANES, _LANES)],
                                  zero)
        acc_v[...] = acc_v[...] + val
        return carry

    lax.fori_loop(0, _RPW, body, 0)
    pltpu.sync_copy(acc_v, out_hbm.at[pl.ds(wid * _LANES, _LANES)])


def _sc_gather(predicted_target, target):
    mesh = plsc.VectorSubcoreMesh(core_axis_name="c", subcore_axis_name="s")
    call = functools.partial(
        pl.kernel,
        mesh=mesh,
        out_type=jax.ShapeDtypeStruct((_NW * _LANES,), jnp.float32),
        scratch_types=[
            pltpu.VMEM((_RPW + _LANES,), jnp.int32),
            pltpu.VMEM((_RPW + _LANES,), jnp.int32),
            pltpu.VMEM((8, _WIN), jnp.float32),
            pltpu.VMEM((_LANES,), jnp.float32),
        ],
    )(_sc_gather_kernel)
    return call(predicted_target, target)


def kernel(predicted_target, target):
    eps = _SMOOTHING / (_V - 2)
    conf = 1.0 - _SMOOTHING
    c_row = (_V - 2) * eps * math.log(eps) + conf * math.log(conf)

    t_sum, z_sum, n1 = _tc_sums(predicted_target, target.reshape(_N, 1))
    g_parts = _sc_gather(predicted_target, target)

    g_sum = jnp.sum(g_parts)
    loss = (n1[0, 0] * jnp.float32(c_row)
            - jnp.float32(eps) * t_sum[0, 0]
            + jnp.float32(eps) * z_sum[0, 0]
            + jnp.float32(eps - conf) * g_sum)
    return loss
